# baseline (speedup 1.0000x reference)
import functools

import jax
import jax.numpy as jnp
from jax import lax
from jax.experimental import pallas as pl
from jax.experimental.pallas import tpu as pltpu
from jax.experimental.pallas import tpu_sc as plsc

N_NODES = 50000
E_EDGES = 800000
HID = 64
EPS = 1e-5

NW = 32
RANGE = 1568
NPAD = NW * RANGE
ROWS_E = E_EDGES // 128
ROWS_IN = 6256
GPT = 208
WIN = 256
DUMP0 = 800384
EPAD = DUMP0 + 32 * 2048
EROWS_PAD = EPAD // 128
BPAD = 102400

_RANK_ONE = 1

_MESH = dict(core_axis_name="c", subcore_axis_name="s")
_SC_PARAMS = pltpu.CompilerParams(needs_layout_passes=False,
                                  use_tc_tiling_on_sc=False)


def _wid():
    return lax.axis_index("s") * 2 + lax.axis_index("c")


def _bucket(n):
    return lax.shift_right_logical(
        lax.shift_right_logical(n, 5) * 1338, 16)


def _iota16():
    return lax.iota(jnp.int32, 16)


def _hist_body(src_hbm, dst_hbm, hu_hbm, hi_hbm, src_v, dst_v, hu_v, hi_v):
    wid = _wid()
    r0 = lax.shift_right_logical(wid * ROWS_E, 5)
    r1 = lax.shift_right_logical((wid + 1) * ROWS_E, 5)
    r0a = pl.multiple_of(
        lax.shift_left(lax.shift_right_logical(r0, 3), 3), 8)
    d0 = r0 - r0a
    d1 = d0 + (r1 - r0)
    pltpu.sync_copy(src_hbm.at[pl.ds(r0a, GPT)], src_v)
    pltpu.sync_copy(dst_hbm.at[pl.ds(r0a, GPT)], dst_v)
    zeros = jnp.zeros((16,), jnp.int32)
    hu_v[pl.ds(0, 16)] = zeros
    hu_v[pl.ds(16, 16)] = zeros
    hi_v[pl.ds(0, 16)] = zeros
    hi_v[pl.ds(16, 16)] = zeros
    ones = jnp.ones((16,), jnp.int32)

    @pl.loop(d0, d1)
    def _(g):
        for k in range(8):
            s16 = src_v[g, pl.ds(k * 16, 16)]
            d16 = dst_v[g, pl.ds(k * 16, 16)]
            plsc.addupdate_scatter(hu_v, [_bucket(s16)], ones)
            plsc.addupdate_scatter(hi_v, [_bucket(d16)], ones)

    pltpu.sync_copy(hu_v, hu_hbm.at[pl.ds(wid * 32, 32)])
    pltpu.sync_copy(hi_v, hi_hbm.at[pl.ds(wid * 32, 32)])


def _make_hist():
    return pl.kernel(
        _hist_body,
        out_type=[jax.ShapeDtypeStruct((NW * 32,), jnp.int32),
                  jax.ShapeDtypeStruct((NW * 32,), jnp.int32)],
        mesh=plsc.VectorSubcoreMesh(**_MESH),
        compiler_params=_SC_PARAMS,
        scratch_types=[
            pltpu.VMEM((GPT, 128), jnp.int32),
            pltpu.VMEM((GPT, 128), jnp.int32),
            pltpu.VMEM((32,), jnp.int32),
            pltpu.VMEM((32,), jnp.int32),
        ],
    )


def _col_sums(hist_v, upto=None):
    init = (jnp.zeros((16,), jnp.int32), jnp.zeros((16,), jnp.int32))

    @pl.loop(0, 32, init_carry=init)
    def sums(t, carry):
        lo, hi = carry
        row_lo = hist_v[pl.ds(t * 32, 16)]
        row_hi = hist_v[pl.ds(t * 32 + 16, 16)]
        if upto is not None:
            take = (t < upto).astype(jnp.int32)
            row_lo = row_lo * take
            row_hi = row_hi * take
        return lo + row_lo, hi + row_hi

    return sums


def _excl_prefix(lo, hi):
    clo = plsc.cumsum(lo)
    chi = plsc.cumsum(hi)
    tot_lo = jnp.sum(lo)
    return clo - lo, chi - hi + tot_lo


def _permute_body(src_hbm, dst_hbm, hu_hbm, hi_hbm,
                  sbu_hbm, gbu_hbm, sbi_hbm, gbi_hbm,
                  src_v, dst_v, pu_v, pi_v, hist_v, offu_v, offi_v, zero_v,
                  sem):
    wid = _wid()
    r0 = lax.shift_right_logical(wid * ROWS_E, 5)
    r1 = lax.shift_right_logical((wid + 1) * ROWS_E, 5)
    r0a = pl.multiple_of(
        lax.shift_left(lax.shift_right_logical(r0, 3), 3), 8)
    d0 = r0 - r0a
    d1 = d0 + (r1 - r0)
    pltpu.sync_copy(src_hbm.at[pl.ds(r0a, GPT)], src_v)
    pltpu.sync_copy(dst_hbm.at[pl.ds(r0a, GPT)], dst_v)

    @pl.when(wid == 0)
    def _():
        z16 = jnp.zeros((16,), jnp.int32)
        for k in range(24):
            zero_v[pl.ds(k * 16, 16)] = z16
        for hbm in (sbu_hbm, gbu_hbm, sbi_hbm, gbi_hbm):
            pltpu.sync_copy(zero_v, hbm.at[pl.ds(E_EDGES, 384)])

    def run_direction(seg_v, h_hbm, off_v, pos_v):
        pltpu.sync_copy(h_hbm, hist_v)
        lo, hi = _col_sums(hist_v)
        exlo, exhi = _excl_prefix(lo, hi)
        mylo, myhi = _col_sums(hist_v, upto=wid)
        off_v[pl.ds(0, 16)] = exlo + mylo
        off_v[pl.ds(16, 16)] = exhi + myhi

        @pl.loop(d0, d1)
        def _(g):
            for k in range(8):
                s16 = seg_v[g, pl.ds(k * 16, 16)]
                b16 = _bucket(s16)
                rank, lastm = plsc.scan_count(b16)
                base = plsc.load_gather(off_v, [b16])
                pos = base + rank - _RANK_ONE
                plsc.store_scatter(off_v, [b16], pos + 1, mask=lastm)
                pos_v[g, pl.ds(k * 16, 16)] = pos

    run_direction(src_v, hu_hbm, offu_v, pu_v)
    run_direction(dst_v, hi_hbm, offi_v, pi_v)

    dump = DUMP0 + wid * 2048

    def fill_dump(g):
        patt = dump + lax.shift_left(lax.bitwise_and(g, 15), 7) + _iota16()
        for k in range(8):
            pu_v[g, pl.ds(k * 16, 16)] = patt + k * 16
            pi_v[g, pl.ds(k * 16, 16)] = patt + k * 16

    pl.loop(0, d0)(fill_dump)
    pl.loop(d1, GPT)(fill_dump)

    @pl.loop(0, GPT // 8)
    def _(bi):
        descs = []
        for j in range(8):
            g = bi * 8 + j
            descs.append(
                pltpu.async_copy(src_v.at[g], sbu_hbm.at[pu_v.at[g]], sem))
            descs.append(
                pltpu.async_copy(dst_v.at[g], gbu_hbm.at[pu_v.at[g]], sem))
            descs.append(
                pltpu.async_copy(dst_v.at[g], sbi_hbm.at[pi_v.at[g]], sem))
            descs.append(
                pltpu.async_copy(src_v.at[g], gbi_hbm.at[pi_v.at[g]], sem))
        for d in descs:
            d.wait()


def _make_permute():
    eshape = jax.ShapeDtypeStruct((EPAD,), jnp.int32)
    return pl.kernel(
        _permute_body,
        out_type=[eshape, eshape, eshape, eshape],
        mesh=plsc.VectorSubcoreMesh(**_MESH),
        compiler_params=_SC_PARAMS,
        scratch_types=[
            pltpu.VMEM((GPT, 128), jnp.int32),
            pltpu.VMEM((GPT, 128), jnp.int32),
            pltpu.VMEM((GPT, 128), jnp.int32),
            pltpu.VMEM((GPT, 128), jnp.int32),
            pltpu.VMEM((1024,), jnp.int32),
            pltpu.VMEM((32,), jnp.int32),
            pltpu.VMEM((32,), jnp.int32),
            pltpu.VMEM((384,), jnp.int32),
            pltpu.SemaphoreType.DMA,
        ],
    )


def _seg_start_cnt(hist_v, b):
    lo, hi = _col_sums(hist_v)
    io = _iota16()
    start = (jnp.sum(lo * (io < b).astype(jnp.int32)) +
             jnp.sum(hi * ((io + 16) < b).astype(jnp.int32)))
    idx = io * 32 + b
    cnt = (jnp.sum(plsc.load_gather(hist_v, [idx])) +
           jnp.sum(plsc.load_gather(hist_v, [idx + 16 * 32])))
    return start, cnt


def _agg_body(is_max, table_hbm, sb_hbm, gb_hbm, h_hbm, out_hbm,
              acc_v, rows_v, sbw_v, gbw_v, loc_v, cnt_v, hist_v, off_v, sem):
    wid = _wid()
    b = wid
    base = b * RANGE
    pltpu.sync_copy(h_hbm, hist_v)
    start, cnt = _seg_start_cnt(hist_v, b)
    astart = pl.multiple_of(
        lax.shift_left(lax.shift_right_logical(start, 3), 3), 8)
    nwin = lax.shift_right_logical(start + cnt - astart + WIN - 1, 8)

    init = jnp.full((16,), -jnp.inf if is_max else 0.0, jnp.float32)

    @pl.loop(0, (RANGE + 1) * 4)
    def _(r):
        acc_v[pl.ds(r * 16, 16)] = init

    if not is_max:
        zf = jnp.zeros((16,), jnp.float32)

        @pl.loop(0, (RANGE + 32) // 16)
        def _(r):
            cnt_v[pl.ds(r * 16, 16)] = zf
        onesf = jnp.ones((16,), jnp.float32)

    @pl.loop(0, nwin)
    def _(w):
        wbase = astart + w * WIN
        pltpu.sync_copy(sb_hbm.at[pl.ds(wbase, WIN)], sbw_v)
        pltpu.sync_copy(gb_hbm.at[pl.ds(wbase, 128)], gbw_v.at[0])
        pltpu.sync_copy(gb_hbm.at[pl.ds(wbase + 128, 128)], gbw_v.at[1])
        g1 = pltpu.async_copy(table_hbm.at[gbw_v.at[0]],
                              rows_v.at[pl.ds(0, 128)], sem)
        g2 = pltpu.async_copy(table_hbm.at[gbw_v.at[1]],
                              rows_v.at[pl.ds(128, 128)], sem)
        for k in range(16):
            epos = wbase + k * 16 + _iota16()
            s16 = sbw_v[pl.ds(k * 16, 16)]
            valid = (epos >= start) & (epos < start + cnt)
            loc16 = jnp.where(valid, s16 - base, RANGE)
            loc_v[pl.ds(k * 16, 16)] = loc16
            if not is_max:
                plsc.addupdate_scatter(cnt_v, [loc16], onesf)
        g1.wait()
        g2.wait()

        @pl.loop(0, WIN // 16)
        def _(v):
            loc16 = loc_v[pl.ds(v * 16, 16)] * HID
            for i in range(16):
                off = loc16[i]
                e = v * 16 + i
                for k in range(4):
                    a = acc_v[pl.ds(off + k * 16, 16)]
                    m = rows_v[e, pl.ds(k * 16, 16)]
                    acc_v[pl.ds(off + k * 16, 16)] = (
                        jnp.maximum(a, m) if is_max else a + m)

    if is_max:
        @pl.loop(0, RANGE * 4)
        def _(r):
            v = acc_v[pl.ds(r * 16, 16)]
            acc_v[pl.ds(r * 16, 16)] = jnp.where(v == -jnp.inf, 0.0, v)
    else:
        @pl.loop(0, RANGE // 16)
        def _(r16):
            inv16 = 1.0 / jnp.maximum(cnt_v[pl.ds(r16 * 16, 16)], 1.0)
            for i in range(16):
                inv = inv16[i]
                r = r16 * 16 + i
                for k in range(4):
                    acc_v[pl.ds(r * HID + k * 16, 16)] = (
                        acc_v[pl.ds(r * HID + k * 16, 16)] * inv)

    pltpu.sync_copy(acc_v.at[pl.ds(0, RANGE * HID)],
                    out_hbm.at[pl.ds(base * HID, RANGE * HID)])


def _make_agg(is_max):
    return pl.kernel(
        functools.partial(_agg_body, is_max),
        out_type=jax.ShapeDtypeStruct((NPAD * HID,), jnp.float32),
        mesh=plsc.VectorSubcoreMesh(**_MESH),
        compiler_params=_SC_PARAMS,
        scratch_types=[
            pltpu.VMEM(((RANGE + 1) * HID,), jnp.float32),
            pltpu.VMEM((WIN, HID), jnp.float32),
            pltpu.VMEM((WIN,), jnp.int32),
            pltpu.VMEM((2, 128), jnp.int32),
            pltpu.VMEM((WIN,), jnp.int32),
            pltpu.VMEM((RANGE + 32,), jnp.float32),
            pltpu.VMEM((1024,), jnp.int32),
            pltpu.VMEM((32,), jnp.int32),
            pltpu.SemaphoreType.DMA,
        ],
    )


def _head_gather_body(ux_hbm, ix_hbm, e0_hbm, e1_hbm, uh_hbm, ih_hbm,
                      idx_v, rows_v, sem):
    wid = _wid()
    rpw = (BPAD // 128) // NW

    @pl.loop(0, rpw)
    def _(r):
        row = wid * rpw + r
        pltpu.sync_copy(e0_hbm.at[pl.ds(row * 128, 128)], idx_v.at[0])
        pltpu.async_copy(ux_hbm.at[idx_v.at[0]], rows_v, sem).wait()
        pltpu.sync_copy(rows_v, uh_hbm.at[pl.ds(row * 128, 128)])
        pltpu.sync_copy(e1_hbm.at[pl.ds(row * 128, 128)], idx_v.at[0])
        pltpu.async_copy(ix_hbm.at[idx_v.at[0]], rows_v, sem).wait()
        pltpu.sync_copy(rows_v, ih_hbm.at[pl.ds(row * 128, 128)])


def _make_head_gather():
    return pl.kernel(
        _head_gather_body,
        out_type=[jax.ShapeDtypeStruct((BPAD, HID), jnp.float32),
                  jax.ShapeDtypeStruct((BPAD, HID), jnp.float32)],
        mesh=plsc.VectorSubcoreMesh(**_MESH),
        compiler_params=_SC_PARAMS,
        scratch_types=[
            pltpu.VMEM((1, 128), jnp.int32),
            pltpu.VMEM((128, HID), jnp.float32),
            pltpu.SemaphoreType.DMA,
        ],
    )


_BM = 512


def _proj_body(x_ref, w_ref, b_ref, o_ref):
    o_ref[...] = jnp.dot(x_ref[...], w_ref[...],
                         preferred_element_type=jnp.float32) + b_ref[...]


def _proj(x, w, bvec):
    m, kdim = x.shape
    n = w.shape[1]
    grid = (m + _BM - 1) // _BM
    return pl.pallas_call(
        _proj_body,
        grid=(grid,),
        in_specs=[pl.BlockSpec((_BM, kdim), lambda i: (i, 0)),
                  pl.BlockSpec((kdim, n), lambda i: (0, 0)),
                  pl.BlockSpec((1, n), lambda i: (0, 0))],
        out_specs=pl.BlockSpec((_BM, n), lambda i: (i, 0)),
        out_shape=jax.ShapeDtypeStruct((m, n), jnp.float32),
    )(x, w, bvec.reshape(1, n))


def _layer_body(agg_ref, x_ref, wl_ref, wr_ref, b_ref, o_ref):
    acc = jnp.dot(agg_ref[...], wl_ref[...],
                  preferred_element_type=jnp.float32)
    acc += jnp.dot(x_ref[...], wr_ref[...],
                   preferred_element_type=jnp.float32)
    o_ref[...] = jnp.maximum(acc + b_ref[...], 0.0)


def _layer(agg, x, wl, wr, bvec):
    m = x.shape[0]
    grid = (m + _BM - 1) // _BM
    return pl.pallas_call(
        _layer_body,
        grid=(grid,),
        in_specs=[pl.BlockSpec((_BM, HID), lambda i: (i, 0)),
                  pl.BlockSpec((_BM, HID), lambda i: (i, 0)),
                  pl.BlockSpec((HID, HID), lambda i: (0, 0)),
                  pl.BlockSpec((HID, HID), lambda i: (0, 0)),
                  pl.BlockSpec((1, HID), lambda i: (0, 0))],
        out_specs=pl.BlockSpec((_BM, HID), lambda i: (i, 0)),
        out_shape=jax.ShapeDtypeStruct((m, HID), jnp.float32),
    )(agg, x, wl, wr, bvec.reshape(1, HID))


def _head_body(uh_ref, ih_ref, w1u_ref, w1i_ref, b1_ref, w2_ref, b2_ref,
               o_ref):
    h = jnp.dot(uh_ref[...], w1u_ref[...], preferred_element_type=jnp.float32)
    h += jnp.dot(ih_ref[...], w1i_ref[...], preferred_element_type=jnp.float32)
    h = jnp.maximum(h + b1_ref[...], 0.0)
    o_ref[...] = jnp.dot(h, w2_ref[...],
                         preferred_element_type=jnp.float32) + b2_ref[...]


def _head(uh, ih, w1u, w1i, b1, w2p, b2p):
    m = uh.shape[0]
    grid = (m + _BM - 1) // _BM
    return pl.pallas_call(
        _head_body,
        grid=(grid,),
        in_specs=[pl.BlockSpec((_BM, HID), lambda i: (i, 0)),
                  pl.BlockSpec((_BM, HID), lambda i: (i, 0)),
                  pl.BlockSpec((HID, HID), lambda i: (0, 0)),
                  pl.BlockSpec((HID, HID), lambda i: (0, 0)),
                  pl.BlockSpec((1, HID), lambda i: (0, 0)),
                  pl.BlockSpec((HID, 128), lambda i: (0, 0)),
                  pl.BlockSpec((1, 128), lambda i: (0, 0))],
        out_specs=pl.BlockSpec((_BM, 128), lambda i: (i, 0)),
        out_shape=jax.ShapeDtypeStruct((m, 128), jnp.float32),
    )(uh, ih, w1u, w1i, b1.reshape(1, HID), w2p, b2p.reshape(1, 128))


def kernel(user_features, item_features, params, edge_index, edge_label_index):
    p = params
    s = 1.0 / jnp.sqrt(1.0 + jnp.float32(EPS))

    pad_e = ROWS_IN * 128 - E_EDGES
    src = jnp.pad(edge_index[0].astype(jnp.int32), (0, pad_e)).reshape(
        ROWS_IN, 128)
    dst = jnp.pad(edge_index[1].astype(jnp.int32), (0, pad_e)).reshape(
        ROWS_IN, 128)

    hu_flat, hi_flat = _make_hist()(src, dst)
    sbu, gbu, sbi, gbi = _make_permute()(src, dst, hu_flat, hi_flat)

    ux = _proj(user_features, p['user_proj_W'].T, p['user_proj_b'])
    ix = _proj(item_features, p['item_proj_W'].T, p['item_proj_b'])

    seg_max = _make_agg(True)
    seg_mean = _make_agg(False)

    for l in range(2):
        gu = p[f'u_bn_g_{l}'] * s
        gi = p[f'i_bn_g_{l}'] * s
        u_wl = (p[f'u_Wl_{l}'] * gu[:, None]).T
        u_wr = (p[f'u_Wr_{l}'] * gu[:, None]).T
        u_b = p[f'u_bl_{l}'] * gu + p[f'u_bn_b_{l}']
        i_wl = (p[f'i_Wl_{l}'] * gi[:, None]).T
        i_wr = (p[f'i_Wr_{l}'] * gi[:, None]).T
        i_b = p[f'i_bl_{l}'] * gi + p[f'i_bn_b_{l}']

        m_u = seg_max(ix, sbu, gbu, hu_flat).reshape(NPAD, HID)
        s_i = seg_mean(ux, sbi, gbi, hi_flat).reshape(NPAD, HID)

        ux = _layer(m_u, ux, u_wl, u_wr, u_b)
        ix = _layer(s_i, ix, i_wl, i_wr, i_b)

    eli = edge_label_index.astype(jnp.int32)
    nb = eli.shape[1]
    e0 = jnp.pad(eli[0], (0, BPAD - nb))
    e1 = jnp.pad(eli[1], (0, BPAD - nb))
    uh, ih = _make_head_gather()(ux, ix, e0, e1)

    w1u = p['fc1_W'][:, :HID].T
    w1i = p['fc1_W'][:, HID:].T
    w2p = jnp.zeros((HID, 128), jnp.float32).at[:, :4].set(p['fc2_W'].T)
    b2p = jnp.zeros((128,), jnp.float32).at[:4].set(p['fc2_b'])
    out = _head(uh, ih, w1u, w1i, p['fc1_b'], w2p, b2p)
    return out[:nb, :4]


# speedup vs baseline: 2.0953x; 2.0953x over previous
import functools

import jax
import jax.numpy as jnp
from jax import lax
from jax.experimental import pallas as pl
from jax.experimental.pallas import tpu as pltpu
from jax.experimental.pallas import tpu_sc as plsc

N_NODES = 50000
E_EDGES = 800000
HID = 64
EPS = 1e-5

NW = 32
RANGE = 1568
NPAD = NW * RANGE
ROWS_E = E_EDGES // 128
ROWS_IN = 6256
GPT = 208
WIN = 256
DUMP0 = 800384
EPAD = DUMP0 + 32 * 2048
EROWS_PAD = EPAD // 128
BPAD = 102400

_RANK_ONE = 1

_MESH = dict(core_axis_name="c", subcore_axis_name="s")
_SC_PARAMS = pltpu.CompilerParams(needs_layout_passes=False,
                                  use_tc_tiling_on_sc=False)


def _wid():
    return lax.axis_index("s") * 2 + lax.axis_index("c")


def _bucket(n):
    return lax.shift_right_logical(
        lax.shift_right_logical(n, 5) * 1338, 16)


def _iota16():
    return lax.iota(jnp.int32, 16)


def _hist_body(src_hbm, dst_hbm, hu_hbm, hi_hbm, src_v, dst_v, hu_v, hi_v):
    wid = _wid()
    r0 = lax.shift_right_logical(wid * ROWS_E, 5)
    r1 = lax.shift_right_logical((wid + 1) * ROWS_E, 5)
    r0a = pl.multiple_of(
        lax.shift_left(lax.shift_right_logical(r0, 3), 3), 8)
    d0 = r0 - r0a
    d1 = d0 + (r1 - r0)
    pltpu.sync_copy(src_hbm.at[pl.ds(r0a, GPT)], src_v)
    pltpu.sync_copy(dst_hbm.at[pl.ds(r0a, GPT)], dst_v)
    zeros = jnp.zeros((16,), jnp.int32)
    hu_v[pl.ds(0, 16)] = zeros
    hu_v[pl.ds(16, 16)] = zeros
    hi_v[pl.ds(0, 16)] = zeros
    hi_v[pl.ds(16, 16)] = zeros
    ones = jnp.ones((16,), jnp.int32)

    @pl.loop(d0, d1)
    def _(g):
        for k in range(8):
            s16 = src_v[g, pl.ds(k * 16, 16)]
            d16 = dst_v[g, pl.ds(k * 16, 16)]
            plsc.addupdate_scatter(hu_v, [_bucket(s16)], ones)
            plsc.addupdate_scatter(hi_v, [_bucket(d16)], ones)

    pltpu.sync_copy(hu_v, hu_hbm.at[pl.ds(wid * 32, 32)])
    pltpu.sync_copy(hi_v, hi_hbm.at[pl.ds(wid * 32, 32)])


def _make_hist():
    return pl.kernel(
        _hist_body,
        out_type=[jax.ShapeDtypeStruct((NW * 32,), jnp.int32),
                  jax.ShapeDtypeStruct((NW * 32,), jnp.int32)],
        mesh=plsc.VectorSubcoreMesh(**_MESH),
        compiler_params=_SC_PARAMS,
        scratch_types=[
            pltpu.VMEM((GPT, 128), jnp.int32),
            pltpu.VMEM((GPT, 128), jnp.int32),
            pltpu.VMEM((32,), jnp.int32),
            pltpu.VMEM((32,), jnp.int32),
        ],
    )


def _col_sums(hist_v, upto=None):
    init = (jnp.zeros((16,), jnp.int32), jnp.zeros((16,), jnp.int32))

    @pl.loop(0, 32, init_carry=init)
    def sums(t, carry):
        lo, hi = carry
        row_lo = hist_v[pl.ds(t * 32, 16)]
        row_hi = hist_v[pl.ds(t * 32 + 16, 16)]
        if upto is not None:
            take = (t < upto).astype(jnp.int32)
            row_lo = row_lo * take
            row_hi = row_hi * take
        return lo + row_lo, hi + row_hi

    return sums


def _excl_prefix(lo, hi):
    clo = plsc.cumsum(lo)
    chi = plsc.cumsum(hi)
    tot_lo = jnp.sum(lo)
    return clo - lo, chi - hi + tot_lo


def _permute_body(src_hbm, dst_hbm, hu_hbm, hi_hbm,
                  ebu_hbm, ebi_hbm,
                  src_v, dst_v, pu_v, pi_v, hist_v, offu_v, offi_v, int_v,
                  sem):
    wid = _wid()
    r0 = lax.shift_right_logical(wid * ROWS_E, 5)
    r1 = lax.shift_right_logical((wid + 1) * ROWS_E, 5)
    r0a = pl.multiple_of(
        lax.shift_left(lax.shift_right_logical(r0, 3), 3), 8)
    d0 = r0 - r0a
    d1 = d0 + (r1 - r0)
    pltpu.sync_copy(src_hbm.at[pl.ds(r0a, GPT)], src_v)
    pltpu.sync_copy(dst_hbm.at[pl.ds(r0a, GPT)], dst_v)

    @pl.when(wid == 0)
    def _():
        z16 = jnp.zeros((16,), jnp.int32)
        zc0 = jnp.zeros((16,), jnp.int32)
        for r in range(32):
            rows = _iota16() + r * 16
            for c in range(2):
                plsc.store_scatter(int_v, [rows, zc0 + c], z16)
        for j in range(3):
            pltpu.sync_copy(int_v.at[pl.ds(0, 128)],
                            ebu_hbm.at[pl.ds(E_EDGES + j * 128, 128)])
            pltpu.sync_copy(int_v.at[pl.ds(0, 128)],
                            ebi_hbm.at[pl.ds(E_EDGES + j * 128, 128)])

    def run_direction(seg_v, h_hbm, off_v, pos_v):
        pltpu.sync_copy(h_hbm, hist_v)
        lo, hi = _col_sums(hist_v)
        exlo, exhi = _excl_prefix(lo, hi)
        mylo, myhi = _col_sums(hist_v, upto=wid)
        off_v[pl.ds(0, 16)] = exlo + mylo
        off_v[pl.ds(16, 16)] = exhi + myhi

        @pl.loop(d0, d1)
        def _(g):
            for k in range(8):
                s16 = seg_v[g, pl.ds(k * 16, 16)]
                b16 = _bucket(s16)
                rank, lastm = plsc.scan_count(b16)
                base = plsc.load_gather(off_v, [b16])
                pos = base + rank - _RANK_ONE
                plsc.store_scatter(off_v, [b16], pos + 1, mask=lastm)
                pos_v[g, pl.ds(k * 16, 16)] = pos

    run_direction(src_v, hu_hbm, offu_v, pu_v)
    run_direction(dst_v, hi_hbm, offi_v, pi_v)

    dump = DUMP0 + wid * 2048

    def fill_dump(g):
        patt = dump + lax.shift_left(lax.bitwise_and(g, 15), 7) + _iota16()
        for k in range(8):
            pu_v[g, pl.ds(k * 16, 16)] = patt + k * 16
            pi_v[g, pl.ds(k * 16, 16)] = patt + k * 16

    pl.loop(0, d0)(fill_dump)
    pl.loop(d1, GPT)(fill_dump)

    io16 = _iota16() * 16

    @pl.loop(0, GPT // 4)
    def _(bi):
        descs = []
        zc = jnp.zeros((16,), jnp.int32)
        for j in range(4):
            g = bi * 4 + j
            for k in range(8):
                s16 = src_v[g, pl.ds(k * 16, 16)]
                d16 = dst_v[g, pl.ds(k * 16, 16)]
                rows = _iota16() + (j * 128 + k * 16)
                plsc.store_scatter(int_v, [rows, zc], s16)
                plsc.store_scatter(int_v, [rows, zc + 1], d16)
        for j in range(4):
            g = bi * 4 + j
            descs.append(pltpu.async_copy(
                int_v.at[pl.ds(j * 128, 128)], ebu_hbm.at[pu_v.at[g]], sem))
            descs.append(pltpu.async_copy(
                int_v.at[pl.ds(j * 128, 128)], ebi_hbm.at[pi_v.at[g]], sem))
        for d in descs:
            d.wait()


def _make_permute():
    eshape = jax.ShapeDtypeStruct((EPAD, 16), jnp.int32)
    return pl.kernel(
        _permute_body,
        out_type=[eshape, eshape],
        mesh=plsc.VectorSubcoreMesh(**_MESH),
        compiler_params=_SC_PARAMS,
        scratch_types=[
            pltpu.VMEM((GPT, 128), jnp.int32),
            pltpu.VMEM((GPT, 128), jnp.int32),
            pltpu.VMEM((GPT, 128), jnp.int32),
            pltpu.VMEM((GPT, 128), jnp.int32),
            pltpu.VMEM((1024,), jnp.int32),
            pltpu.VMEM((32,), jnp.int32),
            pltpu.VMEM((32,), jnp.int32),
            pltpu.VMEM((512, 16), jnp.int32),
            pltpu.SemaphoreType.DMA,
        ],
    )


def _seg_start_cnt(hist_v, b):
    lo, hi = _col_sums(hist_v)
    io = _iota16()
    start = (jnp.sum(lo * (io < b).astype(jnp.int32)) +
             jnp.sum(hi * ((io + 16) < b).astype(jnp.int32)))
    idx = io * 32 + b
    cnt = (jnp.sum(plsc.load_gather(hist_v, [idx])) +
           jnp.sum(plsc.load_gather(hist_v, [idx + 16 * 32])))
    return start, cnt


def _agg_body(is_max, table_hbm, eb_hbm, h_hbm, out_hbm,
              acc_v, rows_v, win_v, gbw_v, loc_v, cnt_v, hist_v, off_v, sem):
    seg_col = 0 if is_max else 1
    gat_col = 1 - seg_col
    wid = _wid()
    b = wid
    base = b * RANGE
    pltpu.sync_copy(h_hbm, hist_v)
    start, cnt = _seg_start_cnt(hist_v, b)
    astart = pl.multiple_of(
        lax.shift_left(lax.shift_right_logical(start, 3), 3), 8)
    nwin = lax.shift_right_logical(start + cnt - astart + WIN - 1, 8)

    init = jnp.full((16,), -jnp.inf if is_max else 0.0, jnp.float32)

    @pl.loop(0, (RANGE + 1) * 4)
    def _(r):
        acc_v[pl.ds(r * 16, 16)] = init

    if not is_max:
        zf = jnp.zeros((16,), jnp.float32)

        @pl.loop(0, (RANGE + 32) // 16)
        def _(r):
            cnt_v[pl.ds(r * 16, 16)] = zf
        onesf = jnp.ones((16,), jnp.float32)

    sc16 = jnp.full((16,), seg_col, jnp.int32)
    gc16 = jnp.full((16,), gat_col, jnp.int32)

    @pl.loop(0, nwin)
    def _(w):
        wbase = astart + w * WIN
        pltpu.sync_copy(eb_hbm.at[pl.ds(wbase, WIN)], win_v)
        for k in range(16):
            rows16 = k * 16 + _iota16()
            g16 = plsc.load_gather(win_v, [rows16, gc16])
            gbw_v[k // 8, pl.ds((k % 8) * 16, 16)] = g16
        g1 = pltpu.async_copy(table_hbm.at[gbw_v.at[0]],
                              rows_v.at[pl.ds(0, 128)], sem)
        g2 = pltpu.async_copy(table_hbm.at[gbw_v.at[1]],
                              rows_v.at[pl.ds(128, 128)], sem)
        for k in range(16):
            rows16 = k * 16 + _iota16()
            epos = wbase + rows16
            s16 = plsc.load_gather(win_v, [rows16, sc16])
            valid = (epos >= start) & (epos < start + cnt)
            loc16 = jnp.where(valid, s16 - base, RANGE)
            loc_v[pl.ds(k * 16, 16)] = loc16
            if not is_max:
                plsc.addupdate_scatter(cnt_v, [loc16], onesf)
        g1.wait()
        g2.wait()

        @pl.loop(0, WIN // 16)
        def _(v):
            loc16 = loc_v[pl.ds(v * 16, 16)] * HID
            for i in range(16):
                off = loc16[i]
                e = v * 16 + i
                for k in range(4):
                    a = acc_v[pl.ds(off + k * 16, 16)]
                    m = rows_v[e, pl.ds(k * 16, 16)]
                    acc_v[pl.ds(off + k * 16, 16)] = (
                        jnp.maximum(a, m) if is_max else a + m)

    if is_max:
        @pl.loop(0, RANGE * 4)
        def _(r):
            v = acc_v[pl.ds(r * 16, 16)]
            acc_v[pl.ds(r * 16, 16)] = jnp.where(v == -jnp.inf, 0.0, v)
    else:
        @pl.loop(0, RANGE // 16)
        def _(r16):
            inv16 = 1.0 / jnp.maximum(cnt_v[pl.ds(r16 * 16, 16)], 1.0)
            for i in range(16):
                inv = inv16[i]
                r = r16 * 16 + i
                for k in range(4):
                    acc_v[pl.ds(r * HID + k * 16, 16)] = (
                        acc_v[pl.ds(r * HID + k * 16, 16)] * inv)

    pltpu.sync_copy(acc_v.at[pl.ds(0, RANGE * HID)],
                    out_hbm.at[pl.ds(base * HID, RANGE * HID)])


def _make_agg(is_max):
    return pl.kernel(
        functools.partial(_agg_body, is_max),
        out_type=jax.ShapeDtypeStruct((NPAD * HID,), jnp.float32),
        mesh=plsc.VectorSubcoreMesh(**_MESH),
        compiler_params=_SC_PARAMS,
        scratch_types=[
            pltpu.VMEM(((RANGE + 1) * HID,), jnp.float32),
            pltpu.VMEM((WIN, HID), jnp.float32),
            pltpu.VMEM((WIN, 16), jnp.int32),
            pltpu.VMEM((2, 128), jnp.int32),
            pltpu.VMEM((WIN,), jnp.int32),
            pltpu.VMEM((RANGE + 32,), jnp.float32),
            pltpu.VMEM((1024,), jnp.int32),
            pltpu.VMEM((32,), jnp.int32),
            pltpu.SemaphoreType.DMA,
        ],
    )


def _head_gather_body(ux_hbm, ix_hbm, e0_hbm, e1_hbm, uh_hbm, ih_hbm,
                      idx_v, rows_v, sem):
    wid = _wid()
    rpw = (BPAD // 128) // NW

    @pl.loop(0, rpw)
    def _(r):
        row = wid * rpw + r
        pltpu.sync_copy(e0_hbm.at[pl.ds(row * 128, 128)], idx_v.at[0])
        pltpu.async_copy(ux_hbm.at[idx_v.at[0]], rows_v, sem).wait()
        pltpu.sync_copy(rows_v, uh_hbm.at[pl.ds(row * 128, 128)])
        pltpu.sync_copy(e1_hbm.at[pl.ds(row * 128, 128)], idx_v.at[0])
        pltpu.async_copy(ix_hbm.at[idx_v.at[0]], rows_v, sem).wait()
        pltpu.sync_copy(rows_v, ih_hbm.at[pl.ds(row * 128, 128)])


def _make_head_gather():
    return pl.kernel(
        _head_gather_body,
        out_type=[jax.ShapeDtypeStruct((BPAD, HID), jnp.float32),
                  jax.ShapeDtypeStruct((BPAD, HID), jnp.float32)],
        mesh=plsc.VectorSubcoreMesh(**_MESH),
        compiler_params=_SC_PARAMS,
        scratch_types=[
            pltpu.VMEM((1, 128), jnp.int32),
            pltpu.VMEM((128, HID), jnp.float32),
            pltpu.SemaphoreType.DMA,
        ],
    )


_BM = 512


def _proj_body(x_ref, w_ref, b_ref, o_ref):
    o_ref[...] = jnp.dot(x_ref[...], w_ref[...],
                         preferred_element_type=jnp.float32) + b_ref[...]


def _proj(x, w, bvec):
    m, kdim = x.shape
    n = w.shape[1]
    grid = (m + _BM - 1) // _BM
    return pl.pallas_call(
        _proj_body,
        grid=(grid,),
        in_specs=[pl.BlockSpec((_BM, kdim), lambda i: (i, 0)),
                  pl.BlockSpec((kdim, n), lambda i: (0, 0)),
                  pl.BlockSpec((1, n), lambda i: (0, 0))],
        out_specs=pl.BlockSpec((_BM, n), lambda i: (i, 0)),
        out_shape=jax.ShapeDtypeStruct((m, n), jnp.float32),
    )(x, w, bvec.reshape(1, n))


def _layer_body(agg_ref, x_ref, wl_ref, wr_ref, b_ref, o_ref):
    acc = jnp.dot(agg_ref[...], wl_ref[...],
                  preferred_element_type=jnp.float32)
    acc += jnp.dot(x_ref[...], wr_ref[...],
                   preferred_element_type=jnp.float32)
    o_ref[...] = jnp.maximum(acc + b_ref[...], 0.0)


def _layer(agg, x, wl, wr, bvec):
    m = x.shape[0]
    grid = (m + _BM - 1) // _BM
    return pl.pallas_call(
        _layer_body,
        grid=(grid,),
        in_specs=[pl.BlockSpec((_BM, HID), lambda i: (i, 0)),
                  pl.BlockSpec((_BM, HID), lambda i: (i, 0)),
                  pl.BlockSpec((HID, HID), lambda i: (0, 0)),
                  pl.BlockSpec((HID, HID), lambda i: (0, 0)),
                  pl.BlockSpec((1, HID), lambda i: (0, 0))],
        out_specs=pl.BlockSpec((_BM, HID), lambda i: (i, 0)),
        out_shape=jax.ShapeDtypeStruct((m, HID), jnp.float32),
    )(agg, x, wl, wr, bvec.reshape(1, HID))


def _head_body(uh_ref, ih_ref, w1u_ref, w1i_ref, b1_ref, w2_ref, b2_ref,
               o_ref):
    h = jnp.dot(uh_ref[...], w1u_ref[...], preferred_element_type=jnp.float32)
    h += jnp.dot(ih_ref[...], w1i_ref[...], preferred_element_type=jnp.float32)
    h = jnp.maximum(h + b1_ref[...], 0.0)
    o_ref[...] = jnp.dot(h, w2_ref[...],
                         preferred_element_type=jnp.float32) + b2_ref[...]


def _head(uh, ih, w1u, w1i, b1, w2p, b2p):
    m = uh.shape[0]
    grid = (m + _BM - 1) // _BM
    return pl.pallas_call(
        _head_body,
        grid=(grid,),
        in_specs=[pl.BlockSpec((_BM, HID), lambda i: (i, 0)),
                  pl.BlockSpec((_BM, HID), lambda i: (i, 0)),
                  pl.BlockSpec((HID, HID), lambda i: (0, 0)),
                  pl.BlockSpec((HID, HID), lambda i: (0, 0)),
                  pl.BlockSpec((1, HID), lambda i: (0, 0)),
                  pl.BlockSpec((HID, 128), lambda i: (0, 0)),
                  pl.BlockSpec((1, 128), lambda i: (0, 0))],
        out_specs=pl.BlockSpec((_BM, 128), lambda i: (i, 0)),
        out_shape=jax.ShapeDtypeStruct((m, 128), jnp.float32),
    )(uh, ih, w1u, w1i, b1.reshape(1, HID), w2p, b2p.reshape(1, 128))


def kernel(user_features, item_features, params, edge_index, edge_label_index):
    p = params
    s = 1.0 / jnp.sqrt(1.0 + jnp.float32(EPS))

    pad_e = ROWS_IN * 128 - E_EDGES
    src = jnp.pad(edge_index[0].astype(jnp.int32), (0, pad_e)).reshape(
        ROWS_IN, 128)
    dst = jnp.pad(edge_index[1].astype(jnp.int32), (0, pad_e)).reshape(
        ROWS_IN, 128)

    hu_flat, hi_flat = _make_hist()(src, dst)
    ebu, ebi = _make_permute()(src, dst, hu_flat, hi_flat)

    ux = _proj(user_features, p['user_proj_W'].T, p['user_proj_b'])
    ix = _proj(item_features, p['item_proj_W'].T, p['item_proj_b'])

    seg_max = _make_agg(True)
    seg_mean = _make_agg(False)

    for l in range(2):
        gu = p[f'u_bn_g_{l}'] * s
        gi = p[f'i_bn_g_{l}'] * s
        u_wl = (p[f'u_Wl_{l}'] * gu[:, None]).T
        u_wr = (p[f'u_Wr_{l}'] * gu[:, None]).T
        u_b = p[f'u_bl_{l}'] * gu + p[f'u_bn_b_{l}']
        i_wl = (p[f'i_Wl_{l}'] * gi[:, None]).T
        i_wr = (p[f'i_Wr_{l}'] * gi[:, None]).T
        i_b = p[f'i_bl_{l}'] * gi + p[f'i_bn_b_{l}']

        m_u = seg_max(ix, ebu, hu_flat).reshape(NPAD, HID)
        s_i = seg_mean(ux, ebi, hi_flat).reshape(NPAD, HID)

        ux = _layer(m_u, ux, u_wl, u_wr, u_b)
        ix = _layer(s_i, ix, i_wl, i_wr, i_b)

    eli = edge_label_index.astype(jnp.int32)
    nb = eli.shape[1]
    e0 = jnp.pad(eli[0], (0, BPAD - nb))
    e1 = jnp.pad(eli[1], (0, BPAD - nb))
    uh, ih = _make_head_gather()(ux, ix, e0, e1)

    w1u = p['fc1_W'][:, :HID].T
    w1i = p['fc1_W'][:, HID:].T
    w2p = jnp.zeros((HID, 128), jnp.float32).at[:, :4].set(p['fc2_W'].T)
    b2p = jnp.zeros((128,), jnp.float32).at[:4].set(p['fc2_b'])
    out = _head(uh, ih, w1u, w1i, p['fc1_b'], w2p, b2p)
    return out[:nb, :4]


# speedup vs baseline: 2.1417x; 1.0221x over previous
import functools

import jax
import jax.numpy as jnp
from jax import lax
from jax.experimental import pallas as pl
from jax.experimental.pallas import tpu as pltpu
from jax.experimental.pallas import tpu_sc as plsc

N_NODES = 50000
E_EDGES = 800000
HID = 64
EPS = 1e-5

NW = 32
RANGE = 1568
NPAD = NW * RANGE
ROWS_E = E_EDGES // 128
ROWS_IN = 6256
GPT = 208
WIN = 256
DUMP0 = 800384
EPAD = DUMP0 + 32 * 2048
EROWS_PAD = EPAD // 128
BPAD = 102400

_RANK_ONE = 1

_MESH = dict(core_axis_name="c", subcore_axis_name="s")
_SC_PARAMS = pltpu.CompilerParams(needs_layout_passes=False,
                                  use_tc_tiling_on_sc=False)


def _wid():
    return lax.axis_index("s") * 2 + lax.axis_index("c")


def _bucket(n):
    return lax.shift_right_logical(
        lax.shift_right_logical(n, 5) * 1338, 16)


def _iota16():
    return lax.iota(jnp.int32, 16)


def _hist_body(src_hbm, dst_hbm, hu_hbm, hi_hbm, src_v, dst_v, hu_v, hi_v):
    wid = _wid()
    r0 = lax.shift_right_logical(wid * ROWS_E, 5)
    r1 = lax.shift_right_logical((wid + 1) * ROWS_E, 5)
    r0a = pl.multiple_of(
        lax.shift_left(lax.shift_right_logical(r0, 3), 3), 8)
    d0 = r0 - r0a
    d1 = d0 + (r1 - r0)
    pltpu.sync_copy(src_hbm.at[pl.ds(r0a, GPT)], src_v)
    pltpu.sync_copy(dst_hbm.at[pl.ds(r0a, GPT)], dst_v)
    zeros = jnp.zeros((16,), jnp.int32)
    hu_v[pl.ds(0, 16)] = zeros
    hu_v[pl.ds(16, 16)] = zeros
    hi_v[pl.ds(0, 16)] = zeros
    hi_v[pl.ds(16, 16)] = zeros
    ones = jnp.ones((16,), jnp.int32)

    @pl.loop(d0, d1)
    def _(g):
        for k in range(8):
            s16 = src_v[g, pl.ds(k * 16, 16)]
            d16 = dst_v[g, pl.ds(k * 16, 16)]
            plsc.addupdate_scatter(hu_v, [_bucket(s16)], ones)
            plsc.addupdate_scatter(hi_v, [_bucket(d16)], ones)

    pltpu.sync_copy(hu_v, hu_hbm.at[pl.ds(wid * 32, 32)])
    pltpu.sync_copy(hi_v, hi_hbm.at[pl.ds(wid * 32, 32)])


def _make_hist():
    return pl.kernel(
        _hist_body,
        out_type=[jax.ShapeDtypeStruct((NW * 32,), jnp.int32),
                  jax.ShapeDtypeStruct((NW * 32,), jnp.int32)],
        mesh=plsc.VectorSubcoreMesh(**_MESH),
        compiler_params=_SC_PARAMS,
        scratch_types=[
            pltpu.VMEM((GPT, 128), jnp.int32),
            pltpu.VMEM((GPT, 128), jnp.int32),
            pltpu.VMEM((32,), jnp.int32),
            pltpu.VMEM((32,), jnp.int32),
        ],
    )


def _col_sums(hist_v, upto=None):
    init = (jnp.zeros((16,), jnp.int32), jnp.zeros((16,), jnp.int32))

    @pl.loop(0, 32, init_carry=init)
    def sums(t, carry):
        lo, hi = carry
        row_lo = hist_v[pl.ds(t * 32, 16)]
        row_hi = hist_v[pl.ds(t * 32 + 16, 16)]
        if upto is not None:
            take = (t < upto).astype(jnp.int32)
            row_lo = row_lo * take
            row_hi = row_hi * take
        return lo + row_lo, hi + row_hi

    return sums


def _excl_prefix(lo, hi):
    clo = plsc.cumsum(lo)
    chi = plsc.cumsum(hi)
    tot_lo = jnp.sum(lo)
    return clo - lo, chi - hi + tot_lo


def _permute_body(src_hbm, dst_hbm, hu_hbm, hi_hbm,
                  ebu_hbm, ebi_hbm,
                  src_v, dst_v, pu_v, pi_v, hist_v, offu_v, offi_v, int_v,
                  sem):
    wid = _wid()
    r0 = lax.shift_right_logical(wid * ROWS_E, 5)
    r1 = lax.shift_right_logical((wid + 1) * ROWS_E, 5)
    r0a = pl.multiple_of(
        lax.shift_left(lax.shift_right_logical(r0, 3), 3), 8)
    d0 = r0 - r0a
    d1 = d0 + (r1 - r0)
    pltpu.sync_copy(src_hbm.at[pl.ds(r0a, GPT)], src_v)
    pltpu.sync_copy(dst_hbm.at[pl.ds(r0a, GPT)], dst_v)

    @pl.when(wid == 0)
    def _():
        z16 = jnp.zeros((16,), jnp.int32)
        zc0 = jnp.zeros((16,), jnp.int32)
        for r in range(32):
            rows = _iota16() + r * 16
            for c in range(2):
                plsc.store_scatter(int_v, [rows, zc0 + c], z16)
        for j in range(3):
            pltpu.sync_copy(int_v.at[pl.ds(0, 128)],
                            ebu_hbm.at[pl.ds(E_EDGES + j * 128, 128)])
            pltpu.sync_copy(int_v.at[pl.ds(0, 128)],
                            ebi_hbm.at[pl.ds(E_EDGES + j * 128, 128)])

    def run_direction(seg_v, h_hbm, off_v, pos_v):
        pltpu.sync_copy(h_hbm, hist_v)
        lo, hi = _col_sums(hist_v)
        exlo, exhi = _excl_prefix(lo, hi)
        mylo, myhi = _col_sums(hist_v, upto=wid)
        off_v[pl.ds(0, 16)] = exlo + mylo
        off_v[pl.ds(16, 16)] = exhi + myhi

        @pl.loop(d0, d1)
        def _(g):
            for k in range(8):
                s16 = seg_v[g, pl.ds(k * 16, 16)]
                b16 = _bucket(s16)
                rank, lastm = plsc.scan_count(b16)
                base = plsc.load_gather(off_v, [b16])
                pos = base + rank - _RANK_ONE
                plsc.store_scatter(off_v, [b16], pos + 1, mask=lastm)
                pos_v[g, pl.ds(k * 16, 16)] = pos

    run_direction(src_v, hu_hbm, offu_v, pu_v)
    run_direction(dst_v, hi_hbm, offi_v, pi_v)

    dump = DUMP0 + wid * 2048

    def fill_dump(g):
        patt = dump + lax.shift_left(lax.bitwise_and(g, 15), 7) + _iota16()
        for k in range(8):
            pu_v[g, pl.ds(k * 16, 16)] = patt + k * 16
            pi_v[g, pl.ds(k * 16, 16)] = patt + k * 16

    pl.loop(0, d0)(fill_dump)
    pl.loop(d1, GPT)(fill_dump)

    io16 = _iota16() * 16

    @pl.loop(0, GPT // 4)
    def _(bi):
        descs = []
        zc = jnp.zeros((16,), jnp.int32)
        for j in range(4):
            g = bi * 4 + j
            for k in range(8):
                s16 = src_v[g, pl.ds(k * 16, 16)]
                d16 = dst_v[g, pl.ds(k * 16, 16)]
                rows = _iota16() + (j * 128 + k * 16)
                plsc.store_scatter(int_v, [rows, zc], s16)
                plsc.store_scatter(int_v, [rows, zc + 1], d16)
        for j in range(4):
            g = bi * 4 + j
            descs.append(pltpu.async_copy(
                int_v.at[pl.ds(j * 128, 128)], ebu_hbm.at[pu_v.at[g]], sem))
            descs.append(pltpu.async_copy(
                int_v.at[pl.ds(j * 128, 128)], ebi_hbm.at[pi_v.at[g]], sem))
        for d in descs:
            d.wait()


def _make_permute():
    eshape = jax.ShapeDtypeStruct((EPAD, 16), jnp.int32)
    return pl.kernel(
        _permute_body,
        out_type=[eshape, eshape],
        mesh=plsc.VectorSubcoreMesh(**_MESH),
        compiler_params=_SC_PARAMS,
        scratch_types=[
            pltpu.VMEM((GPT, 128), jnp.int32),
            pltpu.VMEM((GPT, 128), jnp.int32),
            pltpu.VMEM((GPT, 128), jnp.int32),
            pltpu.VMEM((GPT, 128), jnp.int32),
            pltpu.VMEM((1024,), jnp.int32),
            pltpu.VMEM((32,), jnp.int32),
            pltpu.VMEM((32,), jnp.int32),
            pltpu.VMEM((512, 16), jnp.int32),
            pltpu.SemaphoreType.DMA,
        ],
    )


def _seg_start_cnt(hist_v, b):
    lo, hi = _col_sums(hist_v)
    io = _iota16()
    start = (jnp.sum(lo * (io < b).astype(jnp.int32)) +
             jnp.sum(hi * ((io + 16) < b).astype(jnp.int32)))
    idx = io * 32 + b
    cnt = (jnp.sum(plsc.load_gather(hist_v, [idx])) +
           jnp.sum(plsc.load_gather(hist_v, [idx + 16 * 32])))
    return start, cnt


def _agg_body(is_max, table_hbm, eb_hbm, h_hbm, out_hbm,
              acc_v, rows0_v, rows1_v, win0_v, win1_v, gbw0_v, gbw1_v,
              loc0_v, loc1_v, cnt_v, hist_v, semA, semB, semC, semD):
    seg_col = 0 if is_max else 1
    gat_col = 1 - seg_col
    W2 = 128
    wid = _wid()
    b = wid
    base = b * RANGE
    pltpu.sync_copy(h_hbm, hist_v)
    start, cnt = _seg_start_cnt(hist_v, b)
    astart = pl.multiple_of(
        lax.shift_left(lax.shift_right_logical(start, 3), 3), 8)
    nwin = lax.shift_right_logical(start + cnt - astart + W2 - 1, 7)

    init = jnp.full((16,), -jnp.inf if is_max else 0.0, jnp.float32)

    @pl.loop(0, (RANGE + 1) * 4)
    def _(r):
        acc_v[pl.ds(r * 16, 16)] = init

    if not is_max:
        zf = jnp.zeros((16,), jnp.float32)

        @pl.loop(0, (RANGE + 32) // 16)
        def _(r):
            cnt_v[pl.ds(r * 16, 16)] = zf
        onesf = jnp.ones((16,), jnp.float32)

    sc16 = jnp.full((16,), seg_col, jnp.int32)
    gc16 = jnp.full((16,), gat_col, jnp.int32)

    def build_idx(win_v, gbw_v):
        for k in range(8):
            rows16 = k * 16 + _iota16()
            gbw_v[pl.ds(k * 16, 16)] = plsc.load_gather(win_v, [rows16, gc16])

    def build_loc(w, win_v, loc_v):
        wb = astart + w * W2
        for k in range(8):
            rows16 = k * 16 + _iota16()
            epos = wb + rows16
            s16 = plsc.load_gather(win_v, [rows16, sc16])
            valid = (epos >= start) & (epos < start + cnt)
            loc16 = jnp.where(valid, s16 - base, RANGE)
            loc_v[pl.ds(k * 16, 16)] = loc16
            if not is_max:
                plsc.addupdate_scatter(cnt_v, [loc16], onesf)

    def rmw(loc_v, rows_v):
        @pl.loop(0, W2 // 16)
        def _(v):
            loc16 = loc_v[pl.ds(v * 16, 16)] * HID
            for i in range(16):
                off = loc16[i]
                e = v * 16 + i
                for k in range(4):
                    a = acc_v[pl.ds(off + k * 16, 16)]
                    m = rows_v[e, pl.ds(k * 16, 16)]
                    acc_v[pl.ds(off + k * 16, 16)] = (
                        jnp.maximum(a, m) if is_max else a + m)

    @pl.loop(0, lax.shift_right_logical(nwin + 1, 1))
    def _(t):
        w0 = t * 2
        w1 = w0 + 1
        have1 = w1 < nwin
        c0 = pltpu.async_copy(
            eb_hbm.at[pl.ds(astart + w0 * W2, W2)], win0_v, semA)

        @pl.when(have1)
        def _():
            pltpu.async_copy(
                eb_hbm.at[pl.ds(astart + w1 * W2, W2)], win1_v, semB)

        c0.wait()
        build_idx(win0_v, gbw0_v)
        g0 = pltpu.async_copy(table_hbm.at[gbw0_v], rows0_v, semC)

        @pl.when(have1)
        def _():
            pltpu.make_async_copy(
                eb_hbm.at[pl.ds(astart + w1 * W2, W2)], win1_v, semB).wait()
            build_idx(win1_v, gbw1_v)
            pltpu.async_copy(table_hbm.at[gbw1_v], rows1_v, semD)

        build_loc(w0, win0_v, loc0_v)

        @pl.when(have1)
        def _():
            build_loc(w1, win1_v, loc1_v)

        g0.wait()
        rmw(loc0_v, rows0_v)

        @pl.when(have1)
        def _():
            pltpu.make_async_copy(
                table_hbm.at[gbw1_v], rows1_v, semD).wait()
            rmw(loc1_v, rows1_v)

    if is_max:
        @pl.loop(0, RANGE * 4)
        def _(r):
            v = acc_v[pl.ds(r * 16, 16)]
            acc_v[pl.ds(r * 16, 16)] = jnp.where(v == -jnp.inf, 0.0, v)
    else:
        @pl.loop(0, RANGE // 16)
        def _(r16):
            inv16 = 1.0 / jnp.maximum(cnt_v[pl.ds(r16 * 16, 16)], 1.0)
            for i in range(16):
                inv = inv16[i]
                r = r16 * 16 + i
                for k in range(4):
                    acc_v[pl.ds(r * HID + k * 16, 16)] = (
                        acc_v[pl.ds(r * HID + k * 16, 16)] * inv)

    pltpu.sync_copy(acc_v.at[pl.ds(0, RANGE * HID)],
                    out_hbm.at[pl.ds(base * HID, RANGE * HID)])


def _make_agg(is_max):
    return pl.kernel(
        functools.partial(_agg_body, is_max),
        out_type=jax.ShapeDtypeStruct((NPAD * HID,), jnp.float32),
        mesh=plsc.VectorSubcoreMesh(**_MESH),
        compiler_params=_SC_PARAMS,
        scratch_types=[
            pltpu.VMEM(((RANGE + 1) * HID,), jnp.float32),
            pltpu.VMEM((128, HID), jnp.float32),
            pltpu.VMEM((128, HID), jnp.float32),
            pltpu.VMEM((128, 16), jnp.int32),
            pltpu.VMEM((128, 16), jnp.int32),
            pltpu.VMEM((128,), jnp.int32),
            pltpu.VMEM((128,), jnp.int32),
            pltpu.VMEM((128,), jnp.int32),
            pltpu.VMEM((128,), jnp.int32),
            pltpu.VMEM((RANGE + 32,), jnp.float32),
            pltpu.VMEM((1024,), jnp.int32),
            pltpu.SemaphoreType.DMA,
            pltpu.SemaphoreType.DMA,
            pltpu.SemaphoreType.DMA,
            pltpu.SemaphoreType.DMA,
        ],
    )


def _head_gather_body(ux_hbm, ix_hbm, e0_hbm, e1_hbm, uh_hbm, ih_hbm,
                      idx_v, rows_v, sem):
    wid = _wid()
    rpw = (BPAD // 128) // NW

    @pl.loop(0, rpw)
    def _(r):
        row = wid * rpw + r
        pltpu.sync_copy(e0_hbm.at[pl.ds(row * 128, 128)], idx_v.at[0])
        pltpu.async_copy(ux_hbm.at[idx_v.at[0]], rows_v, sem).wait()
        pltpu.sync_copy(rows_v, uh_hbm.at[pl.ds(row * 128, 128)])
        pltpu.sync_copy(e1_hbm.at[pl.ds(row * 128, 128)], idx_v.at[0])
        pltpu.async_copy(ix_hbm.at[idx_v.at[0]], rows_v, sem).wait()
        pltpu.sync_copy(rows_v, ih_hbm.at[pl.ds(row * 128, 128)])


def _make_head_gather():
    return pl.kernel(
        _head_gather_body,
        out_type=[jax.ShapeDtypeStruct((BPAD, HID), jnp.float32),
                  jax.ShapeDtypeStruct((BPAD, HID), jnp.float32)],
        mesh=plsc.VectorSubcoreMesh(**_MESH),
        compiler_params=_SC_PARAMS,
        scratch_types=[
            pltpu.VMEM((1, 128), jnp.int32),
            pltpu.VMEM((128, HID), jnp.float32),
            pltpu.SemaphoreType.DMA,
        ],
    )


_BM = 512


def _proj_body(x_ref, w_ref, b_ref, o_ref):
    o_ref[...] = jnp.dot(x_ref[...], w_ref[...],
                         preferred_element_type=jnp.float32) + b_ref[...]


def _proj(x, w, bvec):
    m, kdim = x.shape
    n = w.shape[1]
    grid = (m + _BM - 1) // _BM
    return pl.pallas_call(
        _proj_body,
        grid=(grid,),
        in_specs=[pl.BlockSpec((_BM, kdim), lambda i: (i, 0)),
                  pl.BlockSpec((kdim, n), lambda i: (0, 0)),
                  pl.BlockSpec((1, n), lambda i: (0, 0))],
        out_specs=pl.BlockSpec((_BM, n), lambda i: (i, 0)),
        out_shape=jax.ShapeDtypeStruct((m, n), jnp.float32),
    )(x, w, bvec.reshape(1, n))


def _layer_body(agg_ref, x_ref, wl_ref, wr_ref, b_ref, o_ref):
    acc = jnp.dot(agg_ref[...], wl_ref[...],
                  preferred_element_type=jnp.float32)
    acc += jnp.dot(x_ref[...], wr_ref[...],
                   preferred_element_type=jnp.float32)
    o_ref[...] = jnp.maximum(acc + b_ref[...], 0.0)


def _layer(agg, x, wl, wr, bvec):
    m = x.shape[0]
    grid = (m + _BM - 1) // _BM
    return pl.pallas_call(
        _layer_body,
        grid=(grid,),
        in_specs=[pl.BlockSpec((_BM, HID), lambda i: (i, 0)),
                  pl.BlockSpec((_BM, HID), lambda i: (i, 0)),
                  pl.BlockSpec((HID, HID), lambda i: (0, 0)),
                  pl.BlockSpec((HID, HID), lambda i: (0, 0)),
                  pl.BlockSpec((1, HID), lambda i: (0, 0))],
        out_specs=pl.BlockSpec((_BM, HID), lambda i: (i, 0)),
        out_shape=jax.ShapeDtypeStruct((m, HID), jnp.float32),
    )(agg, x, wl, wr, bvec.reshape(1, HID))


def _head_body(uh_ref, ih_ref, w1u_ref, w1i_ref, b1_ref, w2_ref, b2_ref,
               o_ref):
    h = jnp.dot(uh_ref[...], w1u_ref[...], preferred_element_type=jnp.float32)
    h += jnp.dot(ih_ref[...], w1i_ref[...], preferred_element_type=jnp.float32)
    h = jnp.maximum(h + b1_ref[...], 0.0)
    o_ref[...] = jnp.dot(h, w2_ref[...],
                         preferred_element_type=jnp.float32) + b2_ref[...]


def _head(uh, ih, w1u, w1i, b1, w2p, b2p):
    m = uh.shape[0]
    grid = (m + _BM - 1) // _BM
    return pl.pallas_call(
        _head_body,
        grid=(grid,),
        in_specs=[pl.BlockSpec((_BM, HID), lambda i: (i, 0)),
                  pl.BlockSpec((_BM, HID), lambda i: (i, 0)),
                  pl.BlockSpec((HID, HID), lambda i: (0, 0)),
                  pl.BlockSpec((HID, HID), lambda i: (0, 0)),
                  pl.BlockSpec((1, HID), lambda i: (0, 0)),
                  pl.BlockSpec((HID, 128), lambda i: (0, 0)),
                  pl.BlockSpec((1, 128), lambda i: (0, 0))],
        out_specs=pl.BlockSpec((_BM, 128), lambda i: (i, 0)),
        out_shape=jax.ShapeDtypeStruct((m, 128), jnp.float32),
    )(uh, ih, w1u, w1i, b1.reshape(1, HID), w2p, b2p.reshape(1, 128))


def kernel(user_features, item_features, params, edge_index, edge_label_index):
    p = params
    s = 1.0 / jnp.sqrt(1.0 + jnp.float32(EPS))

    pad_e = ROWS_IN * 128 - E_EDGES
    src = jnp.pad(edge_index[0].astype(jnp.int32), (0, pad_e)).reshape(
        ROWS_IN, 128)
    dst = jnp.pad(edge_index[1].astype(jnp.int32), (0, pad_e)).reshape(
        ROWS_IN, 128)

    hu_flat, hi_flat = _make_hist()(src, dst)
    ebu, ebi = _make_permute()(src, dst, hu_flat, hi_flat)

    ux = _proj(user_features, p['user_proj_W'].T, p['user_proj_b'])
    ix = _proj(item_features, p['item_proj_W'].T, p['item_proj_b'])

    seg_max = _make_agg(True)
    seg_mean = _make_agg(False)

    for l in range(2):
        gu = p[f'u_bn_g_{l}'] * s
        gi = p[f'i_bn_g_{l}'] * s
        u_wl = (p[f'u_Wl_{l}'] * gu[:, None]).T
        u_wr = (p[f'u_Wr_{l}'] * gu[:, None]).T
        u_b = p[f'u_bl_{l}'] * gu + p[f'u_bn_b_{l}']
        i_wl = (p[f'i_Wl_{l}'] * gi[:, None]).T
        i_wr = (p[f'i_Wr_{l}'] * gi[:, None]).T
        i_b = p[f'i_bl_{l}'] * gi + p[f'i_bn_b_{l}']

        m_u = seg_max(ix, ebu, hu_flat).reshape(NPAD, HID)
        s_i = seg_mean(ux, ebi, hi_flat).reshape(NPAD, HID)

        ux = _layer(m_u, ux, u_wl, u_wr, u_b)
        ix = _layer(s_i, ix, i_wl, i_wr, i_b)

    eli = edge_label_index.astype(jnp.int32)
    nb = eli.shape[1]
    e0 = jnp.pad(eli[0], (0, BPAD - nb))
    e1 = jnp.pad(eli[1], (0, BPAD - nb))
    uh, ih = _make_head_gather()(ux, ix, e0, e1)

    w1u = p['fc1_W'][:, :HID].T
    w1i = p['fc1_W'][:, HID:].T
    w2p = jnp.zeros((HID, 128), jnp.float32).at[:, :4].set(p['fc2_W'].T)
    b2p = jnp.zeros((128,), jnp.float32).at[:4].set(p['fc2_b'])
    out = _head(uh, ih, w1u, w1i, p['fc1_b'], w2p, b2p)
    return out[:nb, :4]
